# Initial kernel scaffold; baseline (speedup 1.0000x reference)
import jax
import jax.numpy as jnp
from jax.experimental import pallas as pl


def kernel(positions, node_attrs, charges, shifts, edge_index, batch, W_embed, atomic_energies_p, Wr1_0, Wr2_0, Wr3_0, wsh_0, Wmsg_0, Wprod_0, Wread_0, Wr1_1, Wr2_1, Wr3_1, wsh_1, Wmsg_1, Wprod_1, Wread_1):
    raise NotImplementedError("write your pallas kernel here")


# speedup vs baseline: 1.8173x; 1.8173x over previous
import functools

import jax
import jax.numpy as jnp
from jax import lax
from jax.experimental import pallas as pl
from jax.experimental.pallas import tpu as pltpu
from jax.experimental.pallas import tpu_sc as plsc

N = 100000
E = 1600000
C = 32
NCORES = 2
NSUB = 16
NW = NCORES * NSUB
HALF = N // 2
NPASS = 2
PART = 25024


def _scatter_zeros():
    return jnp.zeros((PART, C), jnp.float32)


def _sc_gather(table, idx, D, CH):
    e_tot = idx.shape[0]
    b_per_w = e_tot // NW
    n_ch = b_per_w // CH
    mesh = plsc.VectorSubcoreMesh(core_axis_name="c", subcore_axis_name="s")

    @functools.partial(
        pl.kernel,
        mesh=mesh,
        out_type=jax.ShapeDtypeStruct((e_tot, D), jnp.float32),
        scratch_types=[
            pltpu.VMEM((CH,), jnp.int32),
            pltpu.VMEM((CH, D), jnp.float32),
            pltpu.SemaphoreType.DMA,
        ],
        compiler_params=pltpu.CompilerParams(use_tc_tiling_on_sc=False),
    )
    def k(table_hbm, idx_hbm, out_hbm, idx_v, rows_v, sem):
        wid = lax.axis_index("s") * NCORES + lax.axis_index("c")
        base = wid * b_per_w

        def body(c, carry):
            off = pl.multiple_of(base + c * CH, 8)
            pltpu.sync_copy(idx_hbm.at[pl.ds(off, CH)], idx_v)
            pltpu.async_copy(table_hbm.at[idx_v], rows_v, sem).wait()
            pltpu.sync_copy(rows_v, out_hbm.at[pl.ds(off, CH)])
            return carry

        lax.fori_loop(0, n_ch, body, 0)

    return k(table, idx)


def _sc_scatter_add(m, dst):
    e_per_sub = E // NSUB
    CH = 2000
    n_ch = e_per_sub // CH
    rows_per_sub = PART // NSUB
    SP = PART + 16
    mesh = plsc.VectorSubcoreMesh(core_axis_name="c", subcore_axis_name="s")

    @functools.partial(
        pl.kernel,
        mesh=mesh,
        out_type=jax.ShapeDtypeStruct((NPASS * NCORES * PART, C), jnp.float32),
        scratch_types=[
            pltpu.VMEM((CH,), jnp.int32),
            pltpu.VMEM((CH,), jnp.int32),
            pltpu.VMEM((CH, C), jnp.float32),
            pltpu.VMEM_SHARED((SP, C), jnp.float32),
        ],
        compiler_params=pltpu.CompilerParams(use_tc_tiling_on_sc=False),
    )
    def k(m_hbm, dst_hbm, zeros_hbm, out_hbm, idx_v, loc_v, m_v, acc_sh):
        cid = lax.axis_index("c")
        sid = lax.axis_index("s")
        z0 = sid * rows_per_sub
        for p in range(NPASS):
            part_base = (cid * NPASS + p) * PART
            pltpu.sync_copy(zeros_hbm.at[pl.ds(z0, rows_per_sub)],
                            acc_sh.at[pl.ds(z0, rows_per_sub)])
            plsc.subcore_barrier()

            def body(ci, carry):
                off = pl.multiple_of(sid * e_per_sub + ci * CH, 8)
                pltpu.sync_copy(dst_hbm.at[pl.ds(off, CH)], idx_v)
                pltpu.sync_copy(m_hbm.at[pl.ds(off, CH)], m_v)

                def ib(j, c2):
                    v = idx_v[pl.ds(j * 16, 16)]
                    adj = v - part_base
                    ok = jnp.logical_and(adj >= 0, adj < PART)
                    loc_v[pl.ds(j * 16, 16)] = jnp.where(ok, adj, PART)
                    return c2

                lax.fori_loop(0, CH // 16, ib, 0)
                pltpu.sync_copy(m_v, acc_sh.at[loc_v], add=True)
                return carry

            lax.fori_loop(0, n_ch, body, 0)
            plsc.subcore_barrier()
            pltpu.sync_copy(acc_sh.at[pl.ds(z0, rows_per_sub)],
                            out_hbm.at[pl.ds(part_base + z0, rows_per_sub)])
            plsc.subcore_barrier()

    return k(m, dst, _scatter_zeros())[:N]


def _silu(x):
    return x * jax.nn.sigmoid(x)


BN = 2000
BE = 3200
BE2 = 6400


def _embed_body(na_ref, w_ref, out_ref):
    out_ref[...] = jnp.dot(na_ref[...], w_ref[...],
                           preferred_element_type=jnp.float32)


def _embed(node_attrs, W_embed):
    Z = node_attrs.shape[1]
    return pl.pallas_call(
        _embed_body,
        grid=(N // BN,),
        in_specs=[
            pl.BlockSpec((BN, Z), lambda i: (i, 0)),
            pl.BlockSpec((Z, C), lambda i: (0, 0)),
        ],
        out_specs=pl.BlockSpec((BN, C), lambda i: (i, 0)),
        out_shape=jax.ShapeDtypeStruct((N, C), jnp.float32),
    )(node_attrs, W_embed)


def _edge_body(srcp_ref, dstp_ref, g0_ref,
               w10_ref, w20_ref, w30_ref, wsh0_ref,
               w11_ref, w21_ref, w31_ref, wsh1_ref,
               m0_ref, f1_ref):
    st = srcp_ref[...]
    dt = dstp_ref[...]
    vt = jnp.transpose(dt - st)
    vx = vt[0:1, :]
    vy = vt[1:2, :]
    vz = vt[2:3, :]
    l2 = vx * vx + vy * vy + vz * vz + 1e-12
    r = jnp.sqrt(l2)
    inv = 1.0 / r
    x = vx * inv
    y = vy * inv
    z = vz * inv
    x2 = x * x
    y2 = y * y
    z2 = z * z
    terms = [
        jnp.ones_like(x),
        1.7320508 * x, 1.7320508 * y, 1.7320508 * z,
        3.8729835 * x * y, 3.8729835 * y * z,
        1.1180340 * (3.0 * z2 - 1.0),
        3.8729835 * x * z, 1.9364917 * (x2 - y2),
        2.0916500 * y * (3.0 * x2 - y2),
        10.246951 * x * y * z,
        1.6201852 * y * (4.0 * z2 - x2 - y2),
        1.3228757 * z * (2.0 * z2 - 3.0 * x2 - 3.0 * y2),
        1.6201852 * x * (4.0 * z2 - x2 - y2),
        5.1234753 * z * (x2 - y2),
        2.0916500 * x * (x2 - 3.0 * y2),
    ]
    yt = jnp.concatenate(terms, axis=0)
    dn = (((0,), (0,)), ((), ()))
    s0 = lax.dot_general(yt, wsh0_ref[...], dn,
                         preferred_element_type=jnp.float32)
    s1 = lax.dot_general(yt, wsh1_ref[...], dn,
                         preferred_element_type=jnp.float32)
    invr = 1.0 / (r + 1e-9)
    scale = 0.6324555320336759
    rows = [scale * jnp.sin((float(n) * jnp.pi / 5.0) * r) * invr
            for n in range(1, 9)]
    eft = jnp.concatenate(rows, axis=0)
    u = jnp.clip(r / 5.0, 0.0, 1.0)
    u5 = u ** 5.0
    u6 = u5 * u
    u7 = u6 * u
    fc = 1.0 - 21.0 * u5 + 35.0 * u6 - 15.0 * u7
    fc = jnp.where(r < 5.0, fc, 0.0)
    eft = eft * fc

    def mlp(w1, w2, w3):
        h = lax.dot_general(eft, w1, dn,
                            preferred_element_type=jnp.float32)
        h = _silu(h)
        h = _silu(jnp.dot(h, w2, preferred_element_type=jnp.float32))
        return jnp.dot(h, w3, preferred_element_type=jnp.float32)

    r0 = mlp(w10_ref[...], w20_ref[...], w30_ref[...])
    r1 = mlp(w11_ref[...], w21_ref[...], w31_ref[...])
    m0_ref[...] = g0_ref[...] * r0 * s0
    f1_ref[...] = r1 * s1


def _edge_dense(srcp, dstp, g0, w10, w20, w30, wsh0, w11, w21, w31, wsh1):
    wspec = lambda a, b: pl.BlockSpec((a, b), lambda i: (0, 0))
    return pl.pallas_call(
        _edge_body,
        grid=(E // BE,),
        in_specs=[
            pl.BlockSpec((BE, 16), lambda i: (i, 0)),
            pl.BlockSpec((BE, 16), lambda i: (i, 0)),
            pl.BlockSpec((BE, C), lambda i: (i, 0)),
            wspec(8, 64), wspec(64, 64), wspec(64, C), wspec(16, 1),
            wspec(8, 64), wspec(64, 64), wspec(64, C), wspec(16, 1),
        ],
        out_specs=[
            pl.BlockSpec((BE, C), lambda i: (i, 0)),
            pl.BlockSpec((BE, C), lambda i: (i, 0)),
        ],
        out_shape=[
            jax.ShapeDtypeStruct((E, C), jnp.float32),
            jax.ShapeDtypeStruct((E, C), jnp.float32),
        ],
    )(srcp, dstp, g0, w10, w20, w30, wsh0, w11, w21, w31, wsh1)


def _mul_body(a_ref, b_ref, o_ref):
    o_ref[...] = a_ref[...] * b_ref[...]


def _mul(a, b):
    return pl.pallas_call(
        _mul_body,
        grid=(E // BE2,),
        in_specs=[
            pl.BlockSpec((BE2, C), lambda i: (i, 0)),
            pl.BlockSpec((BE2, C), lambda i: (i, 0)),
        ],
        out_specs=pl.BlockSpec((BE2, C), lambda i: (i, 0)),
        out_shape=jax.ShapeDtypeStruct((E, C), jnp.float32),
    )(a, b)


def _node_body(agg_ref, nf_ref, wm_ref, wp_ref, wr_ref, nfo_ref, out_ref):
    a = agg_ref[...] * 0.0625
    h = jnp.dot(a, wm_ref[...], preferred_element_type=jnp.float32)
    h = jnp.dot(h, wp_ref[...], preferred_element_type=jnp.float32)
    nf = nf_ref[...] + _silu(h)
    nfo_ref[...] = nf
    out_ref[...] = jnp.dot(nf, wr_ref[...], preferred_element_type=jnp.float32)


def _node(agg, nf, wm, wp, wr):
    wspec = lambda a, b: pl.BlockSpec((a, b), lambda i: (0, 0))
    return pl.pallas_call(
        _node_body,
        grid=(N // BN,),
        in_specs=[
            pl.BlockSpec((BN, C), lambda i: (i, 0)),
            pl.BlockSpec((BN, C), lambda i: (i, 0)),
            wspec(C, C), wspec(C, C), wspec(C, 4),
        ],
        out_specs=[
            pl.BlockSpec((BN, C), lambda i: (i, 0)),
            pl.BlockSpec((BN, 4), lambda i: (i, 0)),
        ],
        out_shape=[
            jax.ShapeDtypeStruct((N, C), jnp.float32),
            jax.ShapeDtypeStruct((N, 4), jnp.float32),
        ],
    )(agg, nf, wm, wp, wr)


def _final_body(na_ref, aep_ref, o0_ref, o1_ref, ch_ref, pos_ref,
                ne_ref, ad_ref, sums_ref):
    ne = jnp.dot(na_ref[...], aep_ref[...],
                 preferred_element_type=jnp.float32)
    ne_ref[...] = ne
    o0 = o0_ref[...]
    o1 = o1_ref[...]
    o = o0 + o1
    ad_ref[...] = o
    dip = o[:, 1:4] + ch_ref[...] * pos_ref[:, 0:3]
    e0p = jnp.sum(ne, axis=0, keepdims=True)
    e1p = jnp.sum(o0[:, 0:1], axis=0, keepdims=True)
    e2p = jnp.sum(o1[:, 0:1], axis=0, keepdims=True)
    dsum = jnp.sum(dip, axis=0, keepdims=True)
    part = jnp.concatenate(
        [e0p, e1p, e2p, dsum, jnp.zeros((1, 2), jnp.float32)], axis=1)

    @pl.when(pl.program_id(0) == 0)
    def _():
        sums_ref[...] = jnp.zeros_like(sums_ref)

    sums_ref[...] += part


def _final(node_attrs, aep, out0, out1, charges, pos16):
    Z = node_attrs.shape[1]
    return pl.pallas_call(
        _final_body,
        grid=(N // BN,),
        in_specs=[
            pl.BlockSpec((BN, Z), lambda i: (i, 0)),
            pl.BlockSpec((Z, 1), lambda i: (0, 0)),
            pl.BlockSpec((BN, 4), lambda i: (i, 0)),
            pl.BlockSpec((BN, 4), lambda i: (i, 0)),
            pl.BlockSpec((BN, 1), lambda i: (i, 0)),
            pl.BlockSpec((BN, 16), lambda i: (i, 0)),
        ],
        out_specs=[
            pl.BlockSpec((BN, 1), lambda i: (i, 0)),
            pl.BlockSpec((BN, 4), lambda i: (i, 0)),
            pl.BlockSpec((1, 8), lambda i: (0, 0)),
        ],
        out_shape=[
            jax.ShapeDtypeStruct((N, 1), jnp.float32),
            jax.ShapeDtypeStruct((N, 4), jnp.float32),
            jax.ShapeDtypeStruct((1, 8), jnp.float32),
        ],
    )(node_attrs, aep, out0, out1, charges, pos16)


def kernel(positions, node_attrs, charges, shifts, edge_index, batch,
           W_embed, atomic_energies_p,
           Wr1_0, Wr2_0, Wr3_0, wsh_0, Wmsg_0, Wprod_0, Wread_0,
           Wr1_1, Wr2_1, Wr3_1, wsh_1, Wmsg_1, Wprod_1, Wread_1):
    positions = positions.astype(jnp.float32)
    pos16 = jnp.pad(positions, ((0, 0), (0, 13)))
    src = edge_index[0].astype(jnp.int32)
    dst = edge_index[1].astype(jnp.int32)
    wsh0 = wsh_0.reshape(16, 1).astype(jnp.float32)
    wsh1 = wsh_1.reshape(16, 1).astype(jnp.float32)
    aep = atomic_energies_p.reshape(-1, 1).astype(jnp.float32)
    ch = charges.reshape(N, 1).astype(jnp.float32)

    nf0 = _embed(node_attrs.astype(jnp.float32), W_embed)
    srcp = _sc_gather(pos16, src, 16, 2000)
    dstp = _sc_gather(pos16, dst, 16, 2000)
    g0 = _sc_gather(nf0, src, C, 2000)
    m0, f1 = _edge_dense(srcp, dstp, g0,
                         Wr1_0, Wr2_0, Wr3_0, wsh0,
                         Wr1_1, Wr2_1, Wr3_1, wsh1)
    agg0 = _sc_scatter_add(m0, dst)
    nf1, out0 = _node(agg0, nf0, Wmsg_0, Wprod_0, Wread_0)
    g1 = _sc_gather(nf1, src, C, 2000)
    m1 = _mul(g1, f1)
    agg1 = _sc_scatter_add(m1, dst)
    nf2, out1 = _node(agg1, nf1, Wmsg_1, Wprod_1, Wread_1)
    ne, ad, sums = _final(node_attrs.astype(jnp.float32), aep,
                          out0, out1, ch, pos16)

    contributions = sums[:, 0:3]
    total_energy = jnp.sum(contributions, axis=-1)
    node_energy = ne[:, 0]
    atomic_dipoles = ad[:, 1:4]
    total_dipole = sums[:, 3:6]
    return total_energy, node_energy, contributions, total_dipole, atomic_dipoles
